# baseline (speedup 1.0000x reference)
import functools

import jax
import jax.numpy as jnp
from jax import lax
from jax.experimental import pallas as pl
from jax.experimental.pallas import tpu as pltpu
from jax.experimental.pallas import tpu_sc as plsc

HEADS = 32
NUM_REL = 3969
NUM_REL_PAD = 4096
NC, NS, L = 2, 16, 16
NW = NC * NS
HT = 2
NG = HEADS // HT
NQ = NW // NG
RR = 16


def _sc_gather(tableT, rel_index, n):
    rows_q = n // NQ
    CH = RR * n
    nch = rows_q // RR
    vpr = n // L
    assert nch % 2 == 0
    mesh = plsc.VectorSubcoreMesh(
        core_axis_name="c", subcore_axis_name="s", num_cores=NC, num_subcores=NS
    )

    @functools.partial(
        pl.kernel,
        out_type=jax.ShapeDtypeStruct((HEADS, n, n), jnp.float32),
        mesh=mesh,
        compiler_params=pltpu.CompilerParams(
            needs_layout_passes=False, use_tc_tiling_on_sc=True
        ),
        scratch_types=[
            pltpu.VMEM((HT * NUM_REL_PAD,), jnp.float32),
            pltpu.VMEM((2 * RR, n), jnp.int32),
            pltpu.VMEM((2 * HT * RR, n), jnp.float32),
            pltpu.SemaphoreType.DMA,
            pltpu.SemaphoreType.DMA,
            pltpu.SemaphoreType.DMA,
            pltpu.SemaphoreType.DMA,
        ],
    )
    def k(tableT_hbm, rel_hbm, out_hbm, tab_v, idx_v, out_v, is0, is1, os0, os1):
        wid = lax.axis_index("s") * NC + lax.axis_index("c")
        g = wid % NG
        q = wid // NG
        h0 = g * HT
        row0 = q * rows_q

        isems, osems = (is0, is1), (os0, os1)

        def idx_copy(a, b):
            return pltpu.make_async_copy(
                rel_hbm.at[pl.ds(row0 + a * RR, RR), :],
                idx_v.at[pl.ds(b * RR, RR), :],
                isems[b],
            )

        def out_drain(b):
            return pltpu.make_async_copy(
                out_hbm.at[0, pl.ds(0, HT * RR), :],
                out_v.at[pl.ds(b * HT * RR, HT * RR), :],
                osems[b],
            )

        idx_copy(0, 0).start()
        idx_copy(1, 1).start()
        tab_descs = [
            pltpu.async_copy(
                tableT_hbm.at[h0 + r],
                tab_v.at[pl.ds(r * NUM_REL_PAD, NUM_REL_PAD)],
                os0,
            )
            for r in range(HT)
        ]
        for d in tab_descs:
            d.wait()

        def half(t2, a, b):
            idx_copy(a, b).wait()

            @pl.when(t2 > 0)
            def _():
                out_drain(b).wait()

            srow = b * HT * RR

            @plsc.parallel_loop(0, CH // L, unroll=4)
            def gbody(j):
                r = j // vpr
                c = (j % vpr) * L
                iv = idx_v[b * RR + r, pl.ds(c, L)]
                for h in range(HT):
                    out_v[srow + h * RR + r, pl.ds(c, L)] = plsc.load_gather(
                        tab_v, [iv + h * NUM_REL_PAD]
                    )

            for h in range(HT):
                pltpu.async_copy(
                    out_v.at[pl.ds(srow + h * RR, RR), :],
                    out_hbm.at[h0 + h, pl.ds(row0 + a * RR, RR), :],
                    osems[b],
                )

            @pl.when(a + 2 < nch)
            def _():
                idx_copy(a + 2, b).start()

        def body(t2, _):
            half(t2, 2 * t2, 0)
            half(t2, 2 * t2 + 1, 1)
            return _

        lax.fori_loop(0, nch // 2, body, None)
        out_drain(0).wait()
        out_drain(1).wait()

    return k(tableT, rel_index)


def kernel(table, rel_index):
    n = rel_index.shape[0]
    tableT = jnp.pad(table.T, ((0, 0), (0, NUM_REL_PAD - NUM_REL)))
    return _sc_gather(tableT, rel_index, n)


# speedup vs baseline: 1.3069x; 1.3069x over previous
import functools

import jax
import jax.numpy as jnp
from jax import lax
from jax.experimental import pallas as pl
from jax.experimental.pallas import tpu as pltpu
from jax.experimental.pallas import tpu_sc as plsc

HEADS = 32
NUM_REL = 3969
NUM_REL_PAD = 4096
NC, NS, L = 2, 16, 16
NW = NC * NS
HT = 8
NG = HEADS // HT
NQ = NW // NG
RR = 4


def _sc_gather(tableT, rel_index, n):
    rows_q = n // NQ
    CH = RR * n
    nch = rows_q // RR
    vpr = n // L
    assert nch % 2 == 0
    mesh = plsc.VectorSubcoreMesh(
        core_axis_name="c", subcore_axis_name="s", num_cores=NC, num_subcores=NS
    )

    @functools.partial(
        pl.kernel,
        out_type=jax.ShapeDtypeStruct((HEADS, n, n), jnp.float32),
        mesh=mesh,
        compiler_params=pltpu.CompilerParams(
            needs_layout_passes=False, use_tc_tiling_on_sc=True
        ),
        scratch_types=[
            pltpu.VMEM((HT * NUM_REL_PAD,), jnp.float32),
            pltpu.VMEM((2 * RR, n), jnp.int32),
            pltpu.VMEM((2 * HT * RR, n), jnp.float32),
            pltpu.SemaphoreType.DMA,
            pltpu.SemaphoreType.DMA,
            pltpu.SemaphoreType.DMA,
            pltpu.SemaphoreType.DMA,
        ],
    )
    def k(tableT_hbm, rel_hbm, out_hbm, tab_v, idx_v, out_v, is0, is1, os0, os1):
        wid = lax.axis_index("s") * NC + lax.axis_index("c")
        g = wid % NG
        q = wid // NG
        h0 = g * HT
        row0 = q * rows_q

        isems, osems = (is0, is1), (os0, os1)

        def idx_copy(a, b):
            return pltpu.make_async_copy(
                rel_hbm.at[pl.ds(row0 + a * RR, RR), :],
                idx_v.at[pl.ds(b * RR, RR), :],
                isems[b],
            )

        def out_drain(b):
            return pltpu.make_async_copy(
                out_hbm.at[0, pl.ds(0, HT * RR), :],
                out_v.at[pl.ds(b * HT * RR, HT * RR), :],
                osems[b],
            )

        idx_copy(0, 0).start()
        idx_copy(1, 1).start()
        tab_descs = [
            pltpu.async_copy(
                tableT_hbm.at[h0 + r],
                tab_v.at[pl.ds(r * NUM_REL_PAD, NUM_REL_PAD)],
                os0,
            )
            for r in range(HT)
        ]
        for d in tab_descs:
            d.wait()

        def half(t2, a, b):
            idx_copy(a, b).wait()

            @pl.when(t2 > 0)
            def _():
                out_drain(b).wait()

            srow = b * HT * RR

            @plsc.parallel_loop(0, CH // L, unroll=4)
            def gbody(j):
                r = j // vpr
                c = (j % vpr) * L
                iv = idx_v[b * RR + r, pl.ds(c, L)]
                for h in range(HT):
                    out_v[srow + h * RR + r, pl.ds(c, L)] = plsc.load_gather(
                        tab_v, [iv + h * NUM_REL_PAD]
                    )

            for h in range(HT):
                pltpu.async_copy(
                    out_v.at[pl.ds(srow + h * RR, RR), :],
                    out_hbm.at[h0 + h, pl.ds(row0 + a * RR, RR), :],
                    osems[b],
                )

            @pl.when(a + 2 < nch)
            def _():
                idx_copy(a + 2, b).start()

        def body(t2, _):
            half(t2, 2 * t2, 0)
            half(t2, 2 * t2 + 1, 1)
            return _

        lax.fori_loop(0, nch // 2, body, None)
        out_drain(0).wait()
        out_drain(1).wait()

    return k(tableT, rel_index)


def kernel(table, rel_index):
    n = rel_index.shape[0]
    tableT = jnp.pad(table.T, ((0, 0), (0, NUM_REL_PAD - NUM_REL)))
    return _sc_gather(tableT, rel_index, n)
